# speedup vs baseline: 20.2078x; 20.2078x over previous
import functools

import jax
import jax.numpy as jnp
from jax.experimental import pallas as pl
from jax.experimental.pallas import tpu as pltpu

_K = 10
_N = 32768
_ROWS_PER_BLOCK = 8
_NEG_INF = float("-inf")


def _topk_mask_kernel(x_ref, o_ref):
    x = x_ref[...]
    idx = jax.lax.broadcasted_iota(jnp.int32, x.shape, dimension=2)
    work = x
    keep = jnp.zeros(x.shape, dtype=jnp.bool_)
    for _ in range(_K):
        m = jnp.max(work, axis=-1, keepdims=True)
        eq = work == m
        first = jnp.min(jnp.where(eq, idx, _N), axis=-1, keepdims=True)
        hit = idx == first
        keep = jnp.logical_or(keep, hit)
        work = jnp.where(hit, _NEG_INF, work)
    o_ref[...] = jnp.where(keep, x, jnp.zeros_like(x))


@jax.jit
def kernel(inputs):
    B, H, N = inputs.shape
    grid = (B, H // _ROWS_PER_BLOCK)
    spec = pl.BlockSpec((1, _ROWS_PER_BLOCK, N), lambda i, j: (i, j, 0))
    return pl.pallas_call(
        _topk_mask_kernel,
        grid=grid,
        in_specs=[spec],
        out_specs=spec,
        out_shape=jax.ShapeDtypeStruct(inputs.shape, inputs.dtype),
    )(inputs)


# speedup vs baseline: 77.5176x; 3.8360x over previous
import functools

import jax
import jax.numpy as jnp
from jax import lax
from jax.experimental import pallas as pl
from jax.experimental.pallas import tpu as pltpu
from jax.experimental.pallas import tpu_sc as plsc

_B, _H, _N = 32, 32, 32768
_ROWS = _B * _H
_NC, _NS, _L = 2, 16, 16
_NW = _NC * _NS
_RPW = _ROWS // _NW
_NV = _N // _L
_NBLK = _NV // 16
_K = 10
_NEG = float("-inf")


def _merge_top16(t, v):
    sa, _ = plsc.sort_key_val(t, t, descending=False)
    sb, _ = plsc.sort_key_val(v, v, descending=True)
    return jnp.maximum(sa, sb)


def _lane(v, k, iota, fill):
    return jnp.max(jnp.where(iota == k, v, fill))


def _sc_body(x_hbm, o_hbm, row_buf, out_buf, cmax_buf, hot_buf):
    wid = lax.axis_index("s") * _NC + lax.axis_index("c")
    iota = lax.iota(jnp.int32, _L)
    zerov = jnp.zeros((_L,), jnp.float32)
    neginf = jnp.full((_L,), _NEG, jnp.float32)

    def zero_body(i, c):
        out_buf[pl.ds(i * _L, _L)] = zerov
        return c

    lax.fori_loop(0, _NV, zero_body, 0)

    row0 = wid * _RPW

    def row_body(r, carry):
        row = row0 + r
        pltpu.sync_copy(x_hbm.at[row], row_buf)

        def blk_body(i, c):
            base = i * 256
            vs = [row_buf[pl.ds(base + u * _L, _L)] for u in range(16)]
            while len(vs) > 1:
                vs = [jnp.maximum(vs[2 * j], vs[2 * j + 1])
                      for j in range(len(vs) // 2)]
            cmax_buf[pl.ds(i * _L, _L)] = vs[0]
            return c

        lax.fori_loop(0, _NBLK, blk_body, 0)

        def fold_body(k, accs):
            out = []
            for j in range(8):
                cm = cmax_buf[pl.ds((k * 8 + j) * _L, _L)]
                out.append(jnp.maximum(accs[j], cm))
            return tuple(out)

        maccs = lax.fori_loop(0, 16, fold_body, (neginf,) * 8)

        top = maccs[0]
        for j in range(1, 8):
            top = _merge_top16(top, maccs[j])
        tops, _ = plsc.sort_key_val(top, top, descending=True)
        t00 = _lane(tops, _K - 1, iota, neginf)
        t00v = jnp.full((_L,), t00, jnp.float32)

        def hot_body(i, ptr):
            cm = cmax_buf[pl.ds(i * _L, _L)]
            msk = cm >= t00v
            mi = msk.astype(jnp.int32)
            pos = ptr + plsc.cumsum(mi) - 1
            base_vec = i * 256 + iota
            plsc.store_scatter(hot_buf, [pos], base_vec, mask=msk)
            return ptr + jnp.sum(mi)

        nh = lax.fori_loop(0, _NBLK, hot_body, 0)
        ngrp = (nh + _L - 1) // _L
        nhv = jnp.full((_L,), nh, jnp.int32)

        def top_body(g, top):
            hv = hot_buf[pl.ds(g * _L, _L)]
            valid = (iota + g * _L) < nhv
            hv = jnp.where(valid, hv, 0)
            for u in range(16):
                idxv = hv + 16 * u
                v = plsc.load_gather(row_buf, [idxv])
                v = jnp.where(valid, v, neginf)
                top = _merge_top16(top, v)
            return top

        top = lax.fori_loop(0, ngrp, top_body, neginf)
        tsort, _ = plsc.sort_key_val(top, top, descending=True)
        t = _lane(tsort, _K - 1, iota, neginf)
        tv = jnp.full((_L,), t, jnp.float32)

        def gt_body(g, cnt):
            hv = hot_buf[pl.ds(g * _L, _L)]
            valid = (iota + g * _L) < nhv
            hv = jnp.where(valid, hv, 0)
            for u in range(16):
                idxv = hv + 16 * u
                v = plsc.load_gather(row_buf, [idxv])
                m = jnp.logical_and(v > tv, valid)
                plsc.store_scatter(out_buf, [idxv], v, mask=m)
                cnt = cnt + jnp.sum(m.astype(jnp.int32))
            return cnt

        cnt_gt = lax.fori_loop(0, ngrp, gt_body, 0)
        rties = _K - cnt_gt

        def tie_body(k, last):
            lastv = jnp.full((_L,), last, jnp.int32)

            def find_body(g, best):
                hv = hot_buf[pl.ds(g * _L, _L)]
                valid = (iota + g * _L) < nhv
                hv = jnp.where(valid, hv, 0)
                for u in range(16):
                    idxv = hv + 16 * u
                    v = plsc.load_gather(row_buf, [idxv])
                    eq = jnp.logical_and(
                        jnp.logical_and(v == tv, valid), idxv > lastv)
                    cand = jnp.where(eq, idxv, _N)
                    best = jnp.minimum(best, jnp.min(cand))
                return best

            best = lax.fori_loop(0, ngrp, find_body, _N)
            bv = jnp.full((_L,), best, jnp.int32)
            m0 = jnp.logical_and(iota == 0, bv < _N)
            plsc.store_scatter(out_buf, [bv], tv, mask=m0)
            return best

        lax.fori_loop(0, rties, tie_body, -1)

        pltpu.sync_copy(out_buf, o_hbm.at[row])

        def rz_body(g, c):
            hv = hot_buf[pl.ds(g * _L, _L)]
            valid = (iota + g * _L) < nhv
            hv = jnp.where(valid, hv, 0)
            for u in range(16):
                idxv = hv + 16 * u
                plsc.store_scatter(out_buf, [idxv], zerov, mask=valid)
            return c

        lax.fori_loop(0, ngrp, rz_body, 0)
        return carry

    lax.fori_loop(0, _RPW, row_body, 0)


_sc_topk = functools.partial(
    pl.kernel,
    out_type=jax.ShapeDtypeStruct((_ROWS, _N), jnp.float32),
    mesh=plsc.VectorSubcoreMesh(
        core_axis_name="c", subcore_axis_name="s",
        num_cores=_NC, num_subcores=_NS),
    scratch_types=[
        pltpu.VMEM((_N,), jnp.float32),
        pltpu.VMEM((_N,), jnp.float32),
        pltpu.VMEM((_NV,), jnp.float32),
        pltpu.VMEM((_NV,), jnp.int32),
    ],
    compiler_params=pltpu.CompilerParams(needs_layout_passes=False),
)(_sc_body)


@jax.jit
def kernel(inputs):
    x2 = inputs.reshape(_ROWS, _N)
    out = _sc_topk(x2)
    return out.reshape(inputs.shape)


# speedup vs baseline: 120.6646x; 1.5566x over previous
import functools

import jax
import jax.numpy as jnp
from jax import lax
from jax.experimental import pallas as pl
from jax.experimental.pallas import tpu as pltpu
from jax.experimental.pallas import tpu_sc as plsc

_B, _H, _N = 32, 32, 32768
_ROWS = _B * _H
_NC, _NS, _L = 2, 16, 16
_NW = _NC * _NS
_RPW = _ROWS // _NW
_NV = _N // _L
_NBLK = _NV // 16
_K = 10
_NEG = float("-inf")


def _merge_top16(t, v):
    sa, _ = plsc.sort_key_val(t, t, descending=False)
    sb, _ = plsc.sort_key_val(v, v, descending=True)
    return jnp.maximum(sa, sb)


def _lane(v, k, iota, fill):
    return jnp.max(jnp.where(iota == k, v, fill))


def _sc_body(x_hbm, o_hbm, bufs, out_buf, cmax_buf, hot_buf, kepts,
             sems_in, sem_out):
    wid = lax.axis_index("s") * _NC + lax.axis_index("c")
    iota = lax.iota(jnp.int32, _L)
    zerov = jnp.zeros((_L,), jnp.float32)
    zeroi = jnp.zeros((_L,), jnp.int32)
    neginf = jnp.full((_L,), _NEG, jnp.float32)

    row0 = wid * _RPW
    row_last = row0 + _RPW - 1

    def zero_body(i, c):
        out_buf[pl.ds(i * _L, _L)] = zerov
        return c

    lax.fori_loop(0, _NV, zero_body, 0)
    kepts[0][...] = zeroi
    kepts[1][...] = zeroi
    pltpu.async_copy(x_hbm.at[row0], bufs[0], sems_in[0])
    pltpu.async_copy(out_buf, o_hbm.at[row0], sem_out)

    def process(row, cur, nxt, sem_cur, sem_nxt, kept_cur, kept_prev):
        pltpu.make_async_copy(x_hbm.at[row], cur, sem_cur).wait()
        nrow = jnp.minimum(row + 1, row_last)
        pltpu.async_copy(x_hbm.at[nrow], nxt, sem_nxt)

        def blk_body(i, c):
            base = i * 256
            vs = [cur[pl.ds(base + u * _L, _L)] for u in range(16)]
            while len(vs) > 1:
                vs = [jnp.maximum(vs[2 * j], vs[2 * j + 1])
                      for j in range(len(vs) // 2)]
            cmax_buf[pl.ds(i * _L, _L)] = vs[0]
            return c

        lax.fori_loop(0, _NBLK, blk_body, 0)

        def fold_body(k, accs):
            out = []
            for j in range(8):
                cm = cmax_buf[pl.ds((k * 8 + j) * _L, _L)]
                out.append(jnp.maximum(accs[j], cm))
            return tuple(out)

        maccs = lax.fori_loop(0, 16, fold_body, (neginf,) * 8)

        top = maccs[0]
        for j in range(1, 8):
            top = _merge_top16(top, maccs[j])
        tops, _ = plsc.sort_key_val(top, top, descending=True)
        t00 = _lane(tops, _K - 1, iota, neginf)
        t00v = jnp.full((_L,), t00, jnp.float32)

        def hot_body(i, ptr):
            cm = cmax_buf[pl.ds(i * _L, _L)]
            msk = cm >= t00v
            mi = msk.astype(jnp.int32)
            pos = ptr + plsc.cumsum(mi) - 1
            base_vec = i * 256 + iota
            plsc.store_scatter(hot_buf, [pos], base_vec, mask=msk)
            return ptr + jnp.sum(mi)

        nh = lax.fori_loop(0, _NBLK, hot_body, 0)
        ngrp = (nh + _L - 1) // _L
        nhv = jnp.full((_L,), nh, jnp.int32)

        def top_body(g, top):
            hv = hot_buf[pl.ds(g * _L, _L)]
            valid = (iota + g * _L) < nhv
            hv = jnp.where(valid, hv, 0)
            for u in range(16):
                idxv = hv + 16 * u
                v = plsc.load_gather(cur, [idxv])
                v = jnp.where(valid, v, neginf)
                top = _merge_top16(top, v)
            return top

        top = lax.fori_loop(0, ngrp, top_body, neginf)
        tsort, _ = plsc.sort_key_val(top, top, descending=True)
        t = _lane(tsort, _K - 1, iota, neginf)
        tv = jnp.full((_L,), t, jnp.float32)

        pltpu.make_async_copy(out_buf, o_hbm.at[row], sem_out).wait()
        kprev = kept_prev[...]
        plsc.store_scatter(out_buf, [kprev], zerov, mask=iota < _K)

        def gt_body(g, kptr):
            hv = hot_buf[pl.ds(g * _L, _L)]
            valid = (iota + g * _L) < nhv
            hv = jnp.where(valid, hv, 0)
            for u in range(16):
                idxv = hv + 16 * u
                v = plsc.load_gather(cur, [idxv])
                m = jnp.logical_and(v > tv, valid)
                mi = m.astype(jnp.int32)
                pos = kptr + plsc.cumsum(mi) - 1
                plsc.store_scatter(kept_cur, [pos], idxv, mask=m)
                plsc.store_scatter(out_buf, [idxv], v, mask=m)
                kptr = kptr + jnp.sum(mi)
            return kptr

        cnt_gt = lax.fori_loop(0, ngrp, gt_body, 0)

        def tie_body(k, carry):
            last, kptr = carry
            lastv = jnp.full((_L,), last, jnp.int32)

            def find_body(g, best):
                hv = hot_buf[pl.ds(g * _L, _L)]
                valid = (iota + g * _L) < nhv
                hv = jnp.where(valid, hv, 0)
                for u in range(16):
                    idxv = hv + 16 * u
                    v = plsc.load_gather(cur, [idxv])
                    eq = jnp.logical_and(
                        jnp.logical_and(v == tv, valid), idxv > lastv)
                    cand = jnp.where(eq, idxv, _N)
                    best = jnp.minimum(best, jnp.min(cand))
                return best

            best = lax.fori_loop(0, ngrp, find_body, _N)
            bv = jnp.full((_L,), best, jnp.int32)
            m0 = jnp.logical_and(iota == 0, bv < _N)
            plsc.store_scatter(out_buf, [bv], tv, mask=m0)
            plsc.store_scatter(
                kept_cur, [jnp.full((_L,), kptr, jnp.int32)], bv, mask=m0)
            return (best, kptr + 1)

        lax.fori_loop(0, _K - cnt_gt, tie_body, (-1, cnt_gt))

        pltpu.async_copy(out_buf, o_hbm.at[row], sem_out)

    def pair_body(k, c):
        r = row0 + 2 * k
        process(r, bufs[0], bufs[1], sems_in[0], sems_in[1],
                kepts[0], kepts[1])
        process(r + 1, bufs[1], bufs[0], sems_in[1], sems_in[0],
                kepts[1], kepts[0])
        return c

    lax.fori_loop(0, _RPW // 2, pair_body, 0)

    pltpu.make_async_copy(x_hbm.at[row_last], bufs[0], sems_in[0]).wait()
    pltpu.make_async_copy(out_buf, o_hbm.at[row_last], sem_out).wait()


_sc_topk = functools.partial(
    pl.kernel,
    out_type=jax.ShapeDtypeStruct((_ROWS, _N), jnp.float32),
    mesh=plsc.VectorSubcoreMesh(
        core_axis_name="c", subcore_axis_name="s",
        num_cores=_NC, num_subcores=_NS),
    scratch_types=[
        (pltpu.VMEM((_N,), jnp.float32),) * 2,
        pltpu.VMEM((_N,), jnp.float32),
        pltpu.VMEM((_NV,), jnp.float32),
        pltpu.VMEM((_NV,), jnp.int32),
        (pltpu.VMEM((_L,), jnp.int32),) * 2,
        (pltpu.SemaphoreType.DMA,) * 2,
        pltpu.SemaphoreType.DMA,
    ],
    compiler_params=pltpu.CompilerParams(needs_layout_passes=False),
)(_sc_body)


@jax.jit
def kernel(inputs):
    x2 = inputs.reshape(_ROWS, _N)
    out = _sc_topk(x2)
    return out.reshape(inputs.shape)
